# Initial kernel scaffold; baseline (speedup 1.0000x reference)
import jax
import jax.numpy as jnp
from jax.experimental import pallas as pl


def kernel(x, node_coords, edge2nodes, params):
    raise NotImplementedError("write your pallas kernel here")


# speedup vs baseline: 1.4508x; 1.4508x over previous
import functools

import jax
import jax.numpy as jnp
from jax import lax
from jax.experimental import pallas as pl
from jax.experimental.pallas import tpu as pltpu
from jax.experimental.pallas import tpu_sc as plsc

N = 10000
E = 160000
H = 128
D_IN = 768
N_CLASSES = 10

NC = 2
NS = 16
NW = NC * NS
CH = 128
K = 40
E_PAD = NW * K * CH
ROWS_PER_SUB = 632
N_ACC = NS * ROWS_PER_SUB

ET = 2048
NT = 2000

_MESH = plsc.VectorSubcoreMesh(
    core_axis_name="c", subcore_axis_name="s", num_cores=NC, num_subcores=NS)


@functools.partial(
    pl.kernel,
    out_type=(jax.ShapeDtypeStruct((E_PAD, H), jnp.float32),
              jax.ShapeDtypeStruct((E_PAD, H), jnp.float32)),
    mesh=_MESH,
    scratch_types=[
        pltpu.VMEM((K, CH), jnp.int32),
        pltpu.VMEM((CH, H), jnp.float32),
        pltpu.SemaphoreType.DMA,
    ],
)
def _gather_h(tab_hbm, idx_hbm, out0, out1, idxv, rows, sem):
    w = lax.axis_index("s") * NC + lax.axis_index("c")
    for t in range(2):
        out = (out0, out1)[t]
        pltpu.sync_copy(idx_hbm.at[t * NW + w], idxv)

        @pl.loop(0, K)
        def _chunk(j, out=out):
            pltpu.async_copy(tab_hbm.at[idxv.at[j]], rows, sem).wait()
            pltpu.sync_copy(rows, out.at[pl.ds((w * K + j) * CH, CH)])


@functools.partial(
    pl.kernel,
    out_type=jax.ShapeDtypeStruct((2, N_ACC, H), jnp.float32),
    mesh=_MESH,
    scratch_types=[
        pltpu.VMEM((K, CH), jnp.int32),
        pltpu.VMEM((CH, H), jnp.float32),
        pltpu.VMEM_SHARED((N_ACC, H), jnp.float32),
        pltpu.SemaphoreType.DMA,
    ],
)
def _segsum(e_hbm, idx_hbm, z_hbm, out, idxv, rows, acc, sem):
    c = lax.axis_index("c")
    s = lax.axis_index("s")
    w = s * NC + c
    pltpu.sync_copy(z_hbm, acc.at[pl.ds(s * ROWS_PER_SUB, ROWS_PER_SUB)])
    pltpu.sync_copy(idx_hbm.at[NW + w], idxv)
    plsc.subcore_barrier()

    @pl.loop(0, K)
    def _chunk(j):
        pltpu.sync_copy(e_hbm.at[pl.ds((w * K + j) * CH, CH)], rows)
        pltpu.sync_copy(rows, acc.at[idxv.at[j]], add=True)

    plsc.subcore_barrier()
    pltpu.sync_copy(acc.at[pl.ds(s * ROWS_PER_SUB, ROWS_PER_SUB)],
                    out.at[c].at[pl.ds(s * ROWS_PER_SUB, ROWS_PER_SUB)])


def _ln(x, g, be):
    mu = jnp.mean(x, axis=1, keepdims=True)
    xc = x - mu
    var = jnp.mean(xc * xc, axis=1, keepdims=True)
    return xc * lax.rsqrt(var + 1e-5) * g + be


def _bcast(i):
    return (0, 0)


def _edge_mlp(hs, hd, e, W0, b0, W1, b1, W2, b2, g, be):
    def body(hs_ref, hd_ref, e_ref, W0_ref, b0_ref, W1_ref, b1_ref,
             W2_ref, b2_ref, g_ref, be_ref, out_ref):
        i = pl.program_id(0)
        x = (jnp.dot(hs_ref[...], W0_ref[0:H, :], preferred_element_type=jnp.float32)
             + jnp.dot(hd_ref[...], W0_ref[H:2 * H, :], preferred_element_type=jnp.float32)
             + jnp.dot(e_ref[...], W0_ref[2 * H:3 * H, :], preferred_element_type=jnp.float32)
             + b0_ref[...])
        x = jnp.maximum(x, 0.0)
        x = jnp.dot(x, W1_ref[...], preferred_element_type=jnp.float32) + b1_ref[...]
        x = jnp.maximum(x, 0.0)
        x = jnp.dot(x, W2_ref[...], preferred_element_type=jnp.float32) + b2_ref[...]
        y = e_ref[...] + _ln(x, g_ref[...], be_ref[...])
        row = i * ET + lax.broadcasted_iota(jnp.int32, (ET, 1), 0)
        out_ref[...] = jnp.where(row < E, y, 0.0)

    return pl.pallas_call(
        body,
        grid=(E_PAD // ET,),
        in_specs=[pl.BlockSpec((ET, H), lambda i: (i, 0))] * 3 + [
            pl.BlockSpec((3 * H, H), _bcast), pl.BlockSpec((1, H), _bcast),
            pl.BlockSpec((H, H), _bcast), pl.BlockSpec((1, H), _bcast),
            pl.BlockSpec((H, H), _bcast), pl.BlockSpec((1, H), _bcast),
            pl.BlockSpec((1, H), _bcast), pl.BlockSpec((1, H), _bcast)],
        out_specs=pl.BlockSpec((ET, H), lambda i: (i, 0)),
        out_shape=jax.ShapeDtypeStruct((E_PAD, H), jnp.float32),
    )(hs, hd, e, W0, b0, W1, b1, W2, b2, g, be)


def _node_mlp(h, agg2, W0, b0, W1, b1, W2, b2, g, be):
    nt = N // NT

    def body(h_ref, a0_ref, a1_ref, W0_ref, b0_ref, W1_ref, b1_ref,
             W2_ref, b2_ref, g_ref, be_ref, out_ref):
        agg = a0_ref[0] + a1_ref[0]
        x = (jnp.dot(h_ref[...], W0_ref[0:H, :], preferred_element_type=jnp.float32)
             + jnp.dot(agg, W0_ref[H:2 * H, :], preferred_element_type=jnp.float32)
             + b0_ref[...])
        x = jnp.maximum(x, 0.0)
        x = jnp.dot(x, W1_ref[...], preferred_element_type=jnp.float32) + b1_ref[...]
        x = jnp.maximum(x, 0.0)
        x = jnp.dot(x, W2_ref[...], preferred_element_type=jnp.float32) + b2_ref[...]
        out_ref[...] = h_ref[...] + _ln(x, g_ref[...], be_ref[...])

    return pl.pallas_call(
        body,
        grid=(nt,),
        in_specs=[pl.BlockSpec((NT, H), lambda i: (i, 0)),
                  pl.BlockSpec((1, NT, H), lambda i: (0, i, 0)),
                  pl.BlockSpec((1, NT, H), lambda i: (1, i, 0)),
                  pl.BlockSpec((2 * H, H), _bcast), pl.BlockSpec((1, H), _bcast),
                  pl.BlockSpec((H, H), _bcast), pl.BlockSpec((1, H), _bcast),
                  pl.BlockSpec((H, H), _bcast), pl.BlockSpec((1, H), _bcast),
                  pl.BlockSpec((1, H), _bcast), pl.BlockSpec((1, H), _bcast)],
        out_specs=pl.BlockSpec((NT, H), lambda i: (i, 0)),
        out_shape=jax.ShapeDtypeStruct((N, H), jnp.float32),
    )(h, agg2, agg2, W0, b0, W1, b1, W2, b2, g, be)


def _node_encoder(x, W0, b0, W1, b1, W2, b2, g, be):
    nt = N // 1000

    def body(x_ref, W0_ref, b0_ref, W1_ref, b1_ref, W2_ref, b2_ref,
             g_ref, be_ref, out_ref):
        v = jnp.dot(x_ref[...], W0_ref[...], preferred_element_type=jnp.float32) + b0_ref[...]
        v = jnp.maximum(v, 0.0)
        v = jnp.dot(v, W1_ref[...], preferred_element_type=jnp.float32) + b1_ref[...]
        v = jnp.maximum(v, 0.0)
        v = jnp.dot(v, W2_ref[...], preferred_element_type=jnp.float32) + b2_ref[...]
        out_ref[...] = _ln(v, g_ref[...], be_ref[...])

    return pl.pallas_call(
        body,
        grid=(nt,),
        in_specs=[pl.BlockSpec((1000, D_IN), lambda i: (i, 0)),
                  pl.BlockSpec((D_IN, H), _bcast), pl.BlockSpec((1, H), _bcast),
                  pl.BlockSpec((H, H), _bcast), pl.BlockSpec((1, H), _bcast),
                  pl.BlockSpec((H, H), _bcast), pl.BlockSpec((1, H), _bcast),
                  pl.BlockSpec((1, H), _bcast), pl.BlockSpec((1, H), _bcast)],
        out_specs=pl.BlockSpec((1000, H), lambda i: (i, 0)),
        out_shape=jax.ShapeDtypeStruct((N, H), jnp.float32),
    )(x, W0, b0, W1, b1, W2, b2, g, be)


def _edge_encoder(cs, cd, W0, b0, W1, b1, W2, b2, g, be):
    def body(cs_ref, cd_ref, W0_ref, b0_ref, W1_ref, b1_ref, W2_ref, b2_ref,
             g_ref, be_ref, out_ref):
        r0 = cd_ref[:, 0:1] - cs_ref[:, 0:1]
        r1 = cd_ref[:, 1:2] - cs_ref[:, 1:2]
        dist = jnp.sqrt(r0 * r0 + r1 * r1 + 1e-12)
        v = (r0 * W0_ref[0:1, :] + r1 * W0_ref[1:2, :] + dist * W0_ref[2:3, :]
             + b0_ref[...])
        v = jnp.maximum(v, 0.0)
        v = jnp.dot(v, W1_ref[...], preferred_element_type=jnp.float32) + b1_ref[...]
        v = jnp.maximum(v, 0.0)
        v = jnp.dot(v, W2_ref[...], preferred_element_type=jnp.float32) + b2_ref[...]
        out_ref[...] = _ln(v, g_ref[...], be_ref[...])

    return pl.pallas_call(
        body,
        grid=(E_PAD // ET,),
        in_specs=[pl.BlockSpec((ET, H), lambda i: (i, 0)),
                  pl.BlockSpec((ET, H), lambda i: (i, 0)),
                  pl.BlockSpec((3, H), _bcast), pl.BlockSpec((1, H), _bcast),
                  pl.BlockSpec((H, H), _bcast), pl.BlockSpec((1, H), _bcast),
                  pl.BlockSpec((H, H), _bcast), pl.BlockSpec((1, H), _bcast),
                  pl.BlockSpec((1, H), _bcast), pl.BlockSpec((1, H), _bcast)],
        out_specs=pl.BlockSpec((ET, H), lambda i: (i, 0)),
        out_shape=jax.ShapeDtypeStruct((E_PAD, H), jnp.float32),
    )(cs, cd, W0, b0, W1, b1, W2, b2, g, be)


def _decode_pool_cls(h, W0, b0, W1, b1, W2, b2, cW, cb):
    nt = N // NT

    def body(h_ref, W0_ref, b0_ref, W1_ref, b1_ref, W2_ref, b2_ref,
             cW_ref, cb_ref, out_ref, acc_ref):
        i = pl.program_id(0)
        v = jnp.dot(h_ref[...], W0_ref[...], preferred_element_type=jnp.float32) + b0_ref[...]
        v = jnp.maximum(v, 0.0)
        v = jnp.dot(v, W1_ref[...], preferred_element_type=jnp.float32) + b1_ref[...]
        v = jnp.maximum(v, 0.0)
        v = jnp.dot(v, W2_ref[...], preferred_element_type=jnp.float32) + b2_ref[...]
        part = jnp.sum(v, axis=0, keepdims=True)

        @pl.when(i == 0)
        def _init():
            acc_ref[...] = jnp.zeros_like(acc_ref)

        acc_ref[...] += part

        @pl.when(i == nt - 1)
        def _fin():
            pooled = acc_ref[...] * (1.0 / N)
            out_ref[...] = (jnp.dot(pooled, cW_ref[...],
                                    preferred_element_type=jnp.float32)
                            + cb_ref[...])

    return pl.pallas_call(
        body,
        grid=(nt,),
        in_specs=[pl.BlockSpec((NT, H), lambda i: (i, 0)),
                  pl.BlockSpec((H, H), _bcast), pl.BlockSpec((1, H), _bcast),
                  pl.BlockSpec((H, H), _bcast), pl.BlockSpec((1, H), _bcast),
                  pl.BlockSpec((H, H), _bcast), pl.BlockSpec((1, H), _bcast),
                  pl.BlockSpec((H, N_CLASSES), _bcast),
                  pl.BlockSpec((1, N_CLASSES), _bcast)],
        out_specs=pl.BlockSpec((1, N_CLASSES), _bcast),
        out_shape=jax.ShapeDtypeStruct((1, N_CLASSES), jnp.float32),
        scratch_shapes=[pltpu.VMEM((1, H), jnp.float32)],
    )(h, W0, b0, W1, b1, W2, b2, cW, cb)


def _mlp_args(p):
    out = [p["W"][0]]
    out.append(p["b"][0].reshape(1, H))
    out.append(p["W"][1])
    out.append(p["b"][1].reshape(1, H))
    out.append(p["W"][2])
    out.append(p["b"][2].reshape(1, H))
    if "g" in p:
        out.append(p["g"].reshape(1, H))
        out.append(p["be"].reshape(1, H))
    return out


def kernel(x, node_coords, edge2nodes, params):
    idx = edge2nodes.astype(jnp.int32)
    idx3 = jnp.concatenate(
        [idx, jnp.zeros((2, E_PAD - E), jnp.int32)], axis=1
    ).reshape(2 * NW, K, CH)
    coords128 = jnp.zeros((N, H), jnp.float32).at[:, 0:2].set(node_coords)
    zeros_sub = jnp.zeros((ROWS_PER_SUB, H), jnp.float32)

    cs, cd = _gather_h(coords128, idx3)
    e = _edge_encoder(cs, cd, *_mlp_args(params["enc_edge"]))
    h = _node_encoder(x, *_mlp_args(params["enc_node"]))

    xs = {
        "e": [jnp.stack([b["edge"]["W"][0] for b in params["blocks"]]),
              jnp.stack([b["edge"]["b"][0].reshape(1, H) for b in params["blocks"]]),
              jnp.stack([b["edge"]["W"][1] for b in params["blocks"]]),
              jnp.stack([b["edge"]["b"][1].reshape(1, H) for b in params["blocks"]]),
              jnp.stack([b["edge"]["W"][2] for b in params["blocks"]]),
              jnp.stack([b["edge"]["b"][2].reshape(1, H) for b in params["blocks"]]),
              jnp.stack([b["edge"]["g"].reshape(1, H) for b in params["blocks"]]),
              jnp.stack([b["edge"]["be"].reshape(1, H) for b in params["blocks"]])],
        "n": [jnp.stack([b["node"]["W"][0] for b in params["blocks"]]),
              jnp.stack([b["node"]["b"][0].reshape(1, H) for b in params["blocks"]]),
              jnp.stack([b["node"]["W"][1] for b in params["blocks"]]),
              jnp.stack([b["node"]["b"][1].reshape(1, H) for b in params["blocks"]]),
              jnp.stack([b["node"]["W"][2] for b in params["blocks"]]),
              jnp.stack([b["node"]["b"][2].reshape(1, H) for b in params["blocks"]]),
              jnp.stack([b["node"]["g"].reshape(1, H) for b in params["blocks"]]),
              jnp.stack([b["node"]["be"].reshape(1, H) for b in params["blocks"]])],
    }

    def blk(carry, p):
        h, e = carry
        hs, hd = _gather_h(h, idx3)
        e2 = _edge_mlp(hs, hd, e, *p["e"])
        agg2 = _segsum(e2, idx3, zeros_sub)
        h2 = _node_mlp(h, agg2, *p["n"])
        return (h2, e2), None

    (h, e), _ = lax.scan(blk, (h, e), xs)

    out = _decode_pool_cls(h, *_mlp_args(params["dec"]),
                           params["cls_W"], params["cls_b"].reshape(1, N_CLASSES))
    return out.reshape(N_CLASSES)


# speedup vs baseline: 1.9455x; 1.3410x over previous
import functools

import jax
import jax.numpy as jnp
from jax import lax
from jax.experimental import pallas as pl
from jax.experimental.pallas import tpu as pltpu
from jax.experimental.pallas import tpu_sc as plsc

N = 10000
E = 160000
H = 128
D_IN = 768
N_CLASSES = 10

NC = 2
NS = 16
NW = NC * NS
CH = 128
K = 40
E_PAD = NW * K * CH
ROWS_PER_SUB = 632
N_ACC = NS * ROWS_PER_SUB

NB = 5
NBAT = K // NB
NB_S = 2
NBAT_S = K // NB_S

ET = 2048
NT = 2000

_MESH = plsc.VectorSubcoreMesh(
    core_axis_name="c", subcore_axis_name="s", num_cores=NC, num_subcores=NS)


@functools.partial(
    pl.kernel,
    out_type=jax.ShapeDtypeStruct((E_PAD, H), jnp.float32),
    mesh=_MESH,
    scratch_types=[
        pltpu.VMEM((K, CH), jnp.int32),
        pltpu.VMEM((K, CH), jnp.int32),
        pltpu.VMEM((NB, CH, H), jnp.float32),
        pltpu.SemaphoreType.DMA,
        pltpu.SemaphoreType.DMA,
    ],
)
def _gather_mix(tab_hbm, idxa_hbm, idxb_hbm, out, idxa, idxb, bufs, gsem, wsem):
    w = lax.axis_index("s") * NC + lax.axis_index("c")
    pltpu.sync_copy(idxa_hbm.at[w], idxa)
    pltpu.sync_copy(idxb_hbm.at[w], idxb)

    @pl.loop(0, NBAT)
    def _batch(m):
        j0 = m * NB
        for b in range(NB):
            pltpu.async_copy(tab_hbm.at[idxa.at[j0 + b]], bufs.at[b], gsem)
        for b in range(NB):
            pltpu.make_async_copy(tab_hbm.at[idxa.at[0]], bufs.at[b], gsem).wait()
        for b in range(NB):
            pltpu.async_copy(tab_hbm.at[idxb.at[j0 + b]], bufs.at[b], gsem,
                             add=True)
        for b in range(NB):
            pltpu.make_async_copy(tab_hbm.at[idxb.at[0]], bufs.at[b], gsem).wait()
        for b in range(NB):
            pltpu.async_copy(
                bufs.at[b], out.at[pl.ds((w * K + j0 + b) * CH, CH)], wsem)
        for b in range(NB):
            pltpu.make_async_copy(
                bufs.at[b], out.at[pl.ds((w * K + j0 + b) * CH, CH)], wsem).wait()


@functools.partial(
    pl.kernel,
    out_type=jax.ShapeDtypeStruct((2, N_ACC, H), jnp.float32),
    mesh=_MESH,
    scratch_types=[
        pltpu.VMEM((K, CH), jnp.int32),
        pltpu.VMEM((NB_S, CH, H), jnp.float32),
        pltpu.VMEM_SHARED((N_ACC, H), jnp.float32),
        pltpu.SemaphoreType.DMA,
        pltpu.SemaphoreType.DMA,
    ],
)
def _segsum(e_hbm, idx_hbm, z_hbm, out, idxv, bufs, acc, rsem, ssem):
    c = lax.axis_index("c")
    s = lax.axis_index("s")
    w = s * NC + c
    pltpu.sync_copy(z_hbm, acc.at[pl.ds(s * ROWS_PER_SUB, ROWS_PER_SUB)])
    pltpu.sync_copy(idx_hbm.at[w], idxv)
    plsc.subcore_barrier()

    @pl.loop(0, NBAT_S)
    def _batch(m):
        j0 = m * NB_S
        for b in range(NB_S):
            pltpu.async_copy(
                e_hbm.at[pl.ds((w * K + j0 + b) * CH, CH)], bufs.at[b], rsem)
        for b in range(NB_S):
            pltpu.make_async_copy(
                e_hbm.at[pl.ds(0, CH)], bufs.at[b], rsem).wait()
        for b in range(NB_S):
            pltpu.async_copy(bufs.at[b], acc.at[idxv.at[j0 + b]], ssem,
                             add=True)
        for b in range(NB_S):
            pltpu.make_async_copy(bufs.at[b], acc.at[idxv.at[j0 + b]],
                                  ssem).wait()

    plsc.subcore_barrier()
    pltpu.sync_copy(acc.at[pl.ds(s * ROWS_PER_SUB, ROWS_PER_SUB)],
                    out.at[c].at[pl.ds(s * ROWS_PER_SUB, ROWS_PER_SUB)])


def _ln(x, g, be):
    mu = jnp.mean(x, axis=1, keepdims=True)
    xc = x - mu
    var = jnp.mean(xc * xc, axis=1, keepdims=True)
    return xc * lax.rsqrt(var + 1e-5) * g + be


def _bcast(i):
    return (0, 0)


def _premix(h, Wab):
    nt = N // NT

    def body(h_ref, W_ref, out_ref):
        out_ref[...] = jnp.dot(h_ref[...], W_ref[0],
                               preferred_element_type=jnp.float32)

    return pl.pallas_call(
        body,
        grid=(2, nt),
        in_specs=[pl.BlockSpec((NT, H), lambda t, i: (i, 0)),
                  pl.BlockSpec((1, H, H), lambda t, i: (t, 0, 0))],
        out_specs=pl.BlockSpec((NT, H), lambda t, i: (t * (N // NT) + i, 0)),
        out_shape=jax.ShapeDtypeStruct((2 * N, H), jnp.float32),
    )(h, Wab)


def _edge_mlp(x0p, e, W0c, b0, W1, b1, W2, b2, g, be):
    def body(x0p_ref, e_ref, W0c_ref, b0_ref, W1_ref, b1_ref,
             W2_ref, b2_ref, g_ref, be_ref, out_ref):
        i = pl.program_id(0)
        x = (x0p_ref[...]
             + jnp.dot(e_ref[...], W0c_ref[...], preferred_element_type=jnp.float32)
             + b0_ref[...])
        x = jnp.maximum(x, 0.0)
        x = jnp.dot(x, W1_ref[...], preferred_element_type=jnp.float32) + b1_ref[...]
        x = jnp.maximum(x, 0.0)
        x = jnp.dot(x, W2_ref[...], preferred_element_type=jnp.float32) + b2_ref[...]
        y = e_ref[...] + _ln(x, g_ref[...], be_ref[...])
        row = i * ET + lax.broadcasted_iota(jnp.int32, (ET, 1), 0)
        out_ref[...] = jnp.where(row < E, y, 0.0)

    return pl.pallas_call(
        body,
        grid=(E_PAD // ET,),
        in_specs=[pl.BlockSpec((ET, H), lambda i: (i, 0))] * 2 + [
            pl.BlockSpec((H, H), _bcast), pl.BlockSpec((1, H), _bcast),
            pl.BlockSpec((H, H), _bcast), pl.BlockSpec((1, H), _bcast),
            pl.BlockSpec((H, H), _bcast), pl.BlockSpec((1, H), _bcast),
            pl.BlockSpec((1, H), _bcast), pl.BlockSpec((1, H), _bcast)],
        out_specs=pl.BlockSpec((ET, H), lambda i: (i, 0)),
        out_shape=jax.ShapeDtypeStruct((E_PAD, H), jnp.float32),
    )(x0p, e, W0c, b0, W1, b1, W2, b2, g, be)


def _node_mlp(h, agg2, W0, b0, W1, b1, W2, b2, g, be):
    nt = N // NT

    def body(h_ref, a0_ref, a1_ref, W0_ref, b0_ref, W1_ref, b1_ref,
             W2_ref, b2_ref, g_ref, be_ref, out_ref):
        agg = a0_ref[0] + a1_ref[0]
        x = (jnp.dot(h_ref[...], W0_ref[0:H, :], preferred_element_type=jnp.float32)
             + jnp.dot(agg, W0_ref[H:2 * H, :], preferred_element_type=jnp.float32)
             + b0_ref[...])
        x = jnp.maximum(x, 0.0)
        x = jnp.dot(x, W1_ref[...], preferred_element_type=jnp.float32) + b1_ref[...]
        x = jnp.maximum(x, 0.0)
        x = jnp.dot(x, W2_ref[...], preferred_element_type=jnp.float32) + b2_ref[...]
        out_ref[...] = h_ref[...] + _ln(x, g_ref[...], be_ref[...])

    return pl.pallas_call(
        body,
        grid=(nt,),
        in_specs=[pl.BlockSpec((NT, H), lambda i: (i, 0)),
                  pl.BlockSpec((1, NT, H), lambda i: (0, i, 0)),
                  pl.BlockSpec((1, NT, H), lambda i: (1, i, 0)),
                  pl.BlockSpec((2 * H, H), _bcast), pl.BlockSpec((1, H), _bcast),
                  pl.BlockSpec((H, H), _bcast), pl.BlockSpec((1, H), _bcast),
                  pl.BlockSpec((H, H), _bcast), pl.BlockSpec((1, H), _bcast),
                  pl.BlockSpec((1, H), _bcast), pl.BlockSpec((1, H), _bcast)],
        out_specs=pl.BlockSpec((NT, H), lambda i: (i, 0)),
        out_shape=jax.ShapeDtypeStruct((N, H), jnp.float32),
    )(h, agg2, agg2, W0, b0, W1, b1, W2, b2, g, be)


def _node_encoder(x, W0, b0, W1, b1, W2, b2, g, be):
    nt = N // 1000

    def body(x_ref, W0_ref, b0_ref, W1_ref, b1_ref, W2_ref, b2_ref,
             g_ref, be_ref, out_ref):
        v = jnp.dot(x_ref[...], W0_ref[...], preferred_element_type=jnp.float32) + b0_ref[...]
        v = jnp.maximum(v, 0.0)
        v = jnp.dot(v, W1_ref[...], preferred_element_type=jnp.float32) + b1_ref[...]
        v = jnp.maximum(v, 0.0)
        v = jnp.dot(v, W2_ref[...], preferred_element_type=jnp.float32) + b2_ref[...]
        out_ref[...] = _ln(v, g_ref[...], be_ref[...])

    return pl.pallas_call(
        body,
        grid=(nt,),
        in_specs=[pl.BlockSpec((1000, D_IN), lambda i: (i, 0)),
                  pl.BlockSpec((D_IN, H), _bcast), pl.BlockSpec((1, H), _bcast),
                  pl.BlockSpec((H, H), _bcast), pl.BlockSpec((1, H), _bcast),
                  pl.BlockSpec((H, H), _bcast), pl.BlockSpec((1, H), _bcast),
                  pl.BlockSpec((1, H), _bcast), pl.BlockSpec((1, H), _bcast)],
        out_specs=pl.BlockSpec((1000, H), lambda i: (i, 0)),
        out_shape=jax.ShapeDtypeStruct((N, H), jnp.float32),
    )(x, W0, b0, W1, b1, W2, b2, g, be)


def _edge_encoder(rel, W0, b0, W1, b1, W2, b2, g, be):
    def body(rel_ref, W0_ref, b0_ref, W1_ref, b1_ref, W2_ref, b2_ref,
             g_ref, be_ref, out_ref):
        r0 = rel_ref[:, 0:1]
        r1 = rel_ref[:, 1:2]
        dist = jnp.sqrt(r0 * r0 + r1 * r1 + 1e-12)
        v = (r0 * W0_ref[0:1, :] + r1 * W0_ref[1:2, :] + dist * W0_ref[2:3, :]
             + b0_ref[...])
        v = jnp.maximum(v, 0.0)
        v = jnp.dot(v, W1_ref[...], preferred_element_type=jnp.float32) + b1_ref[...]
        v = jnp.maximum(v, 0.0)
        v = jnp.dot(v, W2_ref[...], preferred_element_type=jnp.float32) + b2_ref[...]
        out_ref[...] = _ln(v, g_ref[...], be_ref[...])

    return pl.pallas_call(
        body,
        grid=(E_PAD // ET,),
        in_specs=[pl.BlockSpec((ET, H), lambda i: (i, 0)),
                  pl.BlockSpec((3, H), _bcast), pl.BlockSpec((1, H), _bcast),
                  pl.BlockSpec((H, H), _bcast), pl.BlockSpec((1, H), _bcast),
                  pl.BlockSpec((H, H), _bcast), pl.BlockSpec((1, H), _bcast),
                  pl.BlockSpec((1, H), _bcast), pl.BlockSpec((1, H), _bcast)],
        out_specs=pl.BlockSpec((ET, H), lambda i: (i, 0)),
        out_shape=jax.ShapeDtypeStruct((E_PAD, H), jnp.float32),
    )(rel, W0, b0, W1, b1, W2, b2, g, be)


def _decode_pool_cls(h, W0, b0, W1, b1, W2, b2, cW, cb):
    nt = N // NT

    def body(h_ref, W0_ref, b0_ref, W1_ref, b1_ref, W2_ref, b2_ref,
             cW_ref, cb_ref, out_ref, acc_ref):
        i = pl.program_id(0)
        v = jnp.dot(h_ref[...], W0_ref[...], preferred_element_type=jnp.float32) + b0_ref[...]
        v = jnp.maximum(v, 0.0)
        v = jnp.dot(v, W1_ref[...], preferred_element_type=jnp.float32) + b1_ref[...]
        v = jnp.maximum(v, 0.0)
        v = jnp.dot(v, W2_ref[...], preferred_element_type=jnp.float32) + b2_ref[...]
        part = jnp.sum(v, axis=0, keepdims=True)

        @pl.when(i == 0)
        def _init():
            acc_ref[...] = jnp.zeros_like(acc_ref)

        acc_ref[...] += part

        @pl.when(i == nt - 1)
        def _fin():
            pooled = acc_ref[...] * (1.0 / N)
            out_ref[...] = (jnp.dot(pooled, cW_ref[...],
                                    preferred_element_type=jnp.float32)
                            + cb_ref[...])

    return pl.pallas_call(
        body,
        grid=(nt,),
        in_specs=[pl.BlockSpec((NT, H), lambda i: (i, 0)),
                  pl.BlockSpec((H, H), _bcast), pl.BlockSpec((1, H), _bcast),
                  pl.BlockSpec((H, H), _bcast), pl.BlockSpec((1, H), _bcast),
                  pl.BlockSpec((H, H), _bcast), pl.BlockSpec((1, H), _bcast),
                  pl.BlockSpec((H, N_CLASSES), _bcast),
                  pl.BlockSpec((1, N_CLASSES), _bcast)],
        out_specs=pl.BlockSpec((1, N_CLASSES), _bcast),
        out_shape=jax.ShapeDtypeStruct((1, N_CLASSES), jnp.float32),
        scratch_shapes=[pltpu.VMEM((1, H), jnp.float32)],
    )(h, W0, b0, W1, b1, W2, b2, cW, cb)


def _mlp_args(p):
    out = [p["W"][0]]
    out.append(p["b"][0].reshape(1, H))
    out.append(p["W"][1])
    out.append(p["b"][1].reshape(1, H))
    out.append(p["W"][2])
    out.append(p["b"][2].reshape(1, H))
    if "g" in p:
        out.append(p["g"].reshape(1, H))
        out.append(p["be"].reshape(1, H))
    return out


def _chunked(idx):
    return jnp.concatenate(
        [idx, jnp.zeros((E_PAD - E,), jnp.int32)]).reshape(NW, K, CH)


def kernel(x, node_coords, edge2nodes, params):
    idx = edge2nodes.astype(jnp.int32)
    src3 = _chunked(idx[0])
    dstoff3 = _chunked(idx[1] + N)
    dst3 = _chunked(idx[1])
    zeros_sub = jnp.zeros((ROWS_PER_SUB, H), jnp.float32)

    coords128 = jnp.zeros((N, H), jnp.float32).at[:, 0:2].set(node_coords)
    ctab = jnp.concatenate([-coords128, coords128], axis=0)

    rel = _gather_mix(ctab, src3, dstoff3)
    e = _edge_encoder(rel, *_mlp_args(params["enc_edge"]))
    h = _node_encoder(x, *_mlp_args(params["enc_node"]))

    blocks = params["blocks"]
    xs = {
        "Wab": jnp.stack([
            jnp.stack([b["edge"]["W"][0][0:H, :], b["edge"]["W"][0][H:2 * H, :]])
            for b in blocks]),
        "W0c": jnp.stack([b["edge"]["W"][0][2 * H:3 * H, :] for b in blocks]),
        "e": [jnp.stack([b["edge"]["b"][0].reshape(1, H) for b in blocks]),
              jnp.stack([b["edge"]["W"][1] for b in blocks]),
              jnp.stack([b["edge"]["b"][1].reshape(1, H) for b in blocks]),
              jnp.stack([b["edge"]["W"][2] for b in blocks]),
              jnp.stack([b["edge"]["b"][2].reshape(1, H) for b in blocks]),
              jnp.stack([b["edge"]["g"].reshape(1, H) for b in blocks]),
              jnp.stack([b["edge"]["be"].reshape(1, H) for b in blocks])],
        "n": [jnp.stack([b["node"]["W"][0] for b in blocks]),
              jnp.stack([b["node"]["b"][0].reshape(1, H) for b in blocks]),
              jnp.stack([b["node"]["W"][1] for b in blocks]),
              jnp.stack([b["node"]["b"][1].reshape(1, H) for b in blocks]),
              jnp.stack([b["node"]["W"][2] for b in blocks]),
              jnp.stack([b["node"]["b"][2].reshape(1, H) for b in blocks]),
              jnp.stack([b["node"]["g"].reshape(1, H) for b in blocks]),
              jnp.stack([b["node"]["be"].reshape(1, H) for b in blocks])],
    }

    def blk(carry, p):
        h, e = carry
        tab = _premix(h, p["Wab"])
        x0p = _gather_mix(tab, src3, dstoff3)
        e2 = _edge_mlp(x0p, e, p["W0c"], *p["e"])
        agg2 = _segsum(e2, dst3, zeros_sub)
        h2 = _node_mlp(h, agg2, *p["n"])
        return (h2, e2), None

    (h, e), _ = lax.scan(blk, (h, e), xs)

    out = _decode_pool_cls(h, *_mlp_args(params["dec"]),
                           params["cls_W"], params["cls_b"].reshape(1, N_CLASSES))
    return out.reshape(N_CLASSES)
